# baseline (speedup 1.0000x reference)
import jax
import jax.numpy as jnp
from jax.experimental import pallas as pl

F32 = jnp.float32
BF16 = jnp.bfloat16


def _max3(a, b, c):
    return jnp.maximum(jnp.maximum(a, b), c)


def _pool(v, oh, ow2):
    c = v.shape[2]
    p = v[: 2 * oh].reshape(oh, 2, 2 * ow2, c).max(axis=1)
    q = v[1 : 2 * oh + 1].reshape(oh, 2, 2 * ow2, c)[:, 1]
    hh = jnp.maximum(p, q)
    pv = hh.reshape(oh, ow2, 2, v.shape[2])
    p2 = pv.max(axis=2)
    pp = pv[:, :, 0]
    q2 = jnp.concatenate([pp[:, 1:], pp[:, :1]], axis=1)
    return jnp.maximum(p2, q2)


def _front_body(x_ref, w1_ref, b1_ref, w2_ref, b2_ref, out_ref):
    xf = x_ref[0]
    xcol = jnp.concatenate([xf[0:928], xf[1:929]], axis=1)
    accs = []
    for phase in range(4):
        acc = (jnp.dot(xcol[0:896], w1_ref[phase, 0],
                       preferred_element_type=F32)
               + jnp.dot(xcol[32:928], w1_ref[phase, 1],
                         preferred_element_type=F32))
        accs.append(acc.reshape(28, 32, 128))
    a00, a01, a10, a11 = accs
    p0 = _max3(a00[:27], a10[:27], a00[1:28])
    p1 = _max3(a01[:27], a11[:27], a01[1:28])
    p0r = jnp.concatenate([p0[:, 1:], p0[:, :1]], axis=1)
    h1 = _max3(p0, p1, p0r) + b1_ref[...].reshape(1, 1, 128)
    col = jax.lax.broadcasted_iota(jnp.int32, (27, 32, 128), 1)
    h1 = jnp.where(col < 27, h1.astype(BF16), jnp.zeros((), BF16))
    h1 = h1.reshape(864, 128)
    xext = jnp.concatenate(
        [jnp.zeros((72, 128), BF16), h1, jnp.zeros((72, 128), BF16)], axis=0)
    xcol = jnp.concatenate(
        [xext[6 + dj : 998 + dj, :] for dj in range(5)], axis=1)
    acc = jnp.dot(xcol[0:864], w2_ref[0], preferred_element_type=F32)
    for di in range(1, 5):
        acc = acc + jnp.dot(xcol[32 * di : 32 * di + 864], w2_ref[di],
                            preferred_element_type=F32)
    b = acc.reshape(27, 32, 256) + b2_ref[...].reshape(1, 1, 256)
    out_ref[0] = _pool(b.astype(BF16), 13, 16).reshape(208, 256)


def _conv3x3(v, w_ref, bias, cin):
    r = jax.lax.broadcasted_iota(jnp.int32, (208, cin), 0)
    vm = jnp.where((r % 16) < 13, v, jnp.zeros((), BF16))
    xext = jnp.concatenate(
        [jnp.zeros((24, cin), BF16), vm, jnp.zeros((24, cin), BF16)], axis=0)
    y = bias
    for dj in range(3):
        z = jnp.dot(xext[0:224], w_ref[0, dj], preferred_element_type=F32)
        z = z + jnp.dot(xext[16:240], w_ref[1, dj],
                        preferred_element_type=F32)
        z = z + jnp.dot(xext[32:256], w_ref[2, dj],
                        preferred_element_type=F32)
        y = y + z[7 + dj : 215 + dj]
    return y


def _back_body(x_ref, w3_ref, b3_ref, w4_ref, b4_ref, w5_ref, b5_ref,
               out_ref):
    x = x_ref[0]
    c3 = _conv3x3(x, w3_ref, b3_ref[...].reshape(1, 384), 256)
    c4 = _conv3x3(c3.astype(BF16), w4_ref, b4_ref[...].reshape(1, 384), 384)
    c5 = _conv3x3(c4.astype(BF16), w5_ref, b5_ref[...].reshape(1, 256), 384)
    out_ref[0] = _pool(c5.astype(BF16).reshape(13, 16, 256), 6, 8)


def _fc_body(w_ref, x_ref, b_ref, out_ref):
    out_ref[...] = (
        jnp.dot(w_ref[...].astype(BF16), x_ref[...],
                preferred_element_type=F32)
        + b_ref[...])


def _fc(w, xt, b, n_block):
    n, k = w.shape
    cols = xt.shape[1]
    return pl.pallas_call(
        _fc_body,
        grid=(n // n_block,),
        in_specs=[
            pl.BlockSpec((n_block, k), lambda i: (i, 0)),
            pl.BlockSpec((k, cols), lambda i: (0, 0)),
            pl.BlockSpec((n_block, 1), lambda i: (i, 0)),
        ],
        out_specs=pl.BlockSpec((n_block, cols), lambda i: (i, 0)),
        out_shape=jax.ShapeDtypeStruct((n, cols), F32),
    )(w, xt, b.reshape(n, 1))


@jax.jit
def kernel(x, W1, b1, W2, b2, W3, b3, W4, b4, W5, b5, W6, b6, W7, b7, W8, b8):
    batch = x.shape[0]

    xp = jnp.pad(x.astype(BF16), ((0, 0), (0, 0), (0, 5), (0, 29)))
    xs = xp.reshape(batch, 3, 29, 8, 32, 8).transpose(0, 2, 4, 1, 3, 5)
    xs = jnp.pad(xs.reshape(batch, 928, 192), ((0, 0), (0, 8), (0, 0)))

    w1phases = []
    for pi in (0, 1):
        for pj in (0, 1):
            wp = jnp.pad(W1.astype(BF16),
                         ((0, 32), (0, 0), (4 * pi, 5 - 4 * pi),
                          (4 * pj, 5 - 4 * pj)))
            wp = wp.reshape(128, 3, 2, 8, 2, 8).transpose(2, 4, 1, 3, 5, 0)
            w1phases.append(wp.reshape(2, 384, 128))
    w1s = jnp.stack(w1phases)
    b1p = jnp.pad(b1, (0, 32)).reshape(1, 128)

    w2s = (jnp.pad(W2.astype(BF16), ((0, 0), (0, 32), (0, 0), (0, 0)))
           .transpose(2, 3, 1, 0).reshape(5, 640, 256))

    def conv_w(w):
        return w.astype(BF16).transpose(2, 3, 1, 0)

    w3s, w4s, w5s = conv_w(W3), conv_w(W4), conv_w(W5)

    h = pl.pallas_call(
        _front_body,
        grid=(batch,),
        in_specs=[
            pl.BlockSpec((1, 936, 192), lambda n: (n, 0, 0)),
            pl.BlockSpec((4, 2, 384, 128), lambda n: (0, 0, 0, 0)),
            pl.BlockSpec((1, 128), lambda n: (0, 0)),
            pl.BlockSpec((5, 640, 256), lambda n: (0, 0, 0)),
            pl.BlockSpec((1, 256), lambda n: (0, 0)),
        ],
        out_specs=pl.BlockSpec((1, 208, 256), lambda n: (n, 0, 0)),
        out_shape=jax.ShapeDtypeStruct((batch, 208, 256), BF16),
    )(xs, w1s, b1p, w2s, b2.reshape(1, 256))

    h = pl.pallas_call(
        _back_body,
        grid=(batch,),
        in_specs=[
            pl.BlockSpec((1, 208, 256), lambda n: (n, 0, 0)),
            pl.BlockSpec((3, 3, 256, 384), lambda n: (0, 0, 0, 0)),
            pl.BlockSpec((1, 384), lambda n: (0, 0)),
            pl.BlockSpec((3, 3, 384, 384), lambda n: (0, 0, 0, 0)),
            pl.BlockSpec((1, 384), lambda n: (0, 0)),
            pl.BlockSpec((3, 3, 384, 256), lambda n: (0, 0, 0, 0)),
            pl.BlockSpec((1, 256), lambda n: (0, 0)),
        ],
        out_specs=pl.BlockSpec((1, 6, 8, 256), lambda n: (n, 0, 0, 0)),
        out_shape=jax.ShapeDtypeStruct((batch, 6, 8, 256), BF16),
    )(h, w3s, b3.reshape(1, 384), w4s, b4.reshape(1, 384), w5s,
      b5.reshape(1, 256))

    xt = h[:, :, :6, :].transpose(3, 1, 2, 0).reshape(9216, batch)
    ht = _fc(W6, xt, b6, n_block=512)
    ht = _fc(W7, ht.astype(BF16), b7, n_block=512)
    ht = _fc(W8, ht.astype(BF16), b8, n_block=200)
    return ht.T


# speedup vs baseline: 5.0978x; 5.0978x over previous
import jax
import jax.numpy as jnp
from jax.experimental import pallas as pl

F32 = jnp.float32
BF16 = jnp.bfloat16


def _max3(a, b, c):
    return jnp.maximum(jnp.maximum(a, b), c)


def _pool(v, oh, ow2):
    c = v.shape[2]
    p = v[: 2 * oh].reshape(oh, 2, 2 * ow2, c).max(axis=1)
    q = v[1 : 2 * oh + 1].reshape(oh, 2, 2 * ow2, c)[:, 1]
    hh = jnp.maximum(p, q)
    pv = hh.reshape(oh, ow2, 2, v.shape[2])
    p2 = pv.max(axis=2)
    pp = pv[:, :, 0]
    q2 = jnp.concatenate([pp[:, 1:], pp[:, :1]], axis=1)
    return jnp.maximum(p2, q2)


def _front_body(x_ref, w1_ref, b1_ref, w2_ref, b2_ref, out_ref):
    xf = x_ref[0]
    xcol = jnp.concatenate(
        [xf[dj : dj + 3648, :] for dj in range(3)], axis=1)
    acc = jnp.dot(xcol[0:3520], w1_ref[0], preferred_element_type=F32)
    for di in range(1, 3):
        acc = acc + jnp.dot(xcol[64 * di : 64 * di + 3520], w1_ref[di],
                            preferred_element_type=F32)
    a = acc.reshape(55, 64, 128) + b1_ref[...].reshape(1, 1, 128)
    h1 = _pool(a.astype(BF16), 27, 32)
    col = jax.lax.broadcasted_iota(jnp.int32, (27, 32, 128), 1)
    h1 = jnp.where(col < 27, h1, jnp.zeros((), BF16))
    h1 = h1.reshape(864, 128)
    xext = jnp.concatenate(
        [jnp.zeros((72, 128), BF16), h1, jnp.zeros((72, 128), BF16)], axis=0)
    xcol = jnp.concatenate(
        [xext[6 + dj : 998 + dj, :] for dj in range(5)], axis=1)
    acc = jnp.dot(xcol[0:864], w2_ref[0], preferred_element_type=F32)
    for di in range(1, 5):
        acc = acc + jnp.dot(xcol[32 * di : 32 * di + 864], w2_ref[di],
                            preferred_element_type=F32)
    b = acc.reshape(27, 32, 256) + b2_ref[...].reshape(1, 1, 256)
    out_ref[0] = _pool(b.astype(BF16), 13, 16).reshape(208, 256)


def _conv3x3(v, w_ref, bias, cin):
    r = jax.lax.broadcasted_iota(jnp.int32, (208, cin), 0)
    vm = jnp.where((r % 16) < 13, v, jnp.zeros((), BF16))
    xext = jnp.concatenate(
        [jnp.zeros((24, cin), BF16), vm, jnp.zeros((24, cin), BF16)], axis=0)
    y = bias
    for dj in range(3):
        z = jnp.dot(xext[0:224], w_ref[0, dj], preferred_element_type=F32)
        z = z + jnp.dot(xext[16:240], w_ref[1, dj],
                        preferred_element_type=F32)
        z = z + jnp.dot(xext[32:256], w_ref[2, dj],
                        preferred_element_type=F32)
        y = y + z[7 + dj : 215 + dj]
    return y


def _back_body(x_ref, w3_ref, b3_ref, w4_ref, b4_ref, w5_ref, b5_ref,
               out_ref):
    x = x_ref[0]
    c3 = _conv3x3(x, w3_ref, b3_ref[...].reshape(1, 384), 256)
    c4 = _conv3x3(c3.astype(BF16), w4_ref, b4_ref[...].reshape(1, 384), 384)
    c5 = _conv3x3(c4.astype(BF16), w5_ref, b5_ref[...].reshape(1, 256), 384)
    out_ref[0] = _pool(c5.astype(BF16).reshape(13, 16, 256), 6, 8)


def _fc_body(w_ref, x_ref, b_ref, out_ref):
    out_ref[...] = (
        jnp.dot(w_ref[...].astype(BF16), x_ref[...],
                preferred_element_type=F32)
        + b_ref[...])


def _fc(w, xt, b, n_block):
    n, k = w.shape
    cols = xt.shape[1]
    return pl.pallas_call(
        _fc_body,
        grid=(n // n_block,),
        in_specs=[
            pl.BlockSpec((n_block, k), lambda i: (i, 0)),
            pl.BlockSpec((k, cols), lambda i: (0, 0)),
            pl.BlockSpec((n_block, 1), lambda i: (i, 0)),
        ],
        out_specs=pl.BlockSpec((n_block, cols), lambda i: (i, 0)),
        out_shape=jax.ShapeDtypeStruct((n, cols), F32),
    )(w, xt, b.reshape(n, 1))


@jax.jit
def kernel(x, W1, b1, W2, b2, W3, b3, W4, b4, W5, b5, W6, b6, W7, b7, W8, b8):
    batch = x.shape[0]

    xp = jnp.pad(x.astype(BF16), ((0, 0), (0, 0), (0, 5), (0, 29)))
    xs = xp.reshape(batch, 3, 58, 4, 64, 4).transpose(0, 2, 4, 1, 3, 5)
    xs = xs.reshape(batch, 58 * 64, 48)

    w1p = jnp.pad(W1.astype(BF16), ((0, 32), (0, 0), (0, 1), (0, 1)))
    w1s = w1p.reshape(128, 3, 3, 4, 3, 4).transpose(2, 4, 1, 3, 5, 0)
    w1s = w1s.reshape(3, 3 * 48, 128)
    b1p = jnp.pad(b1, (0, 32)).reshape(1, 128)

    w2s = (jnp.pad(W2.astype(BF16), ((0, 0), (0, 32), (0, 0), (0, 0)))
           .transpose(2, 3, 1, 0).reshape(5, 640, 256))

    def conv_w(w):
        return w.astype(BF16).transpose(2, 3, 1, 0)

    w3s, w4s, w5s = conv_w(W3), conv_w(W4), conv_w(W5)

    h = pl.pallas_call(
        _front_body,
        grid=(batch,),
        in_specs=[
            pl.BlockSpec((1, 58 * 64, 48), lambda n: (n, 0, 0)),
            pl.BlockSpec((3, 3 * 48, 128), lambda n: (0, 0, 0)),
            pl.BlockSpec((1, 128), lambda n: (0, 0)),
            pl.BlockSpec((5, 640, 256), lambda n: (0, 0, 0)),
            pl.BlockSpec((1, 256), lambda n: (0, 0)),
        ],
        out_specs=pl.BlockSpec((1, 208, 256), lambda n: (n, 0, 0)),
        out_shape=jax.ShapeDtypeStruct((batch, 208, 256), BF16),
    )(xs, w1s, b1p, w2s, b2.reshape(1, 256))

    h = pl.pallas_call(
        _back_body,
        grid=(batch,),
        in_specs=[
            pl.BlockSpec((1, 208, 256), lambda n: (n, 0, 0)),
            pl.BlockSpec((3, 3, 256, 384), lambda n: (0, 0, 0, 0)),
            pl.BlockSpec((1, 384), lambda n: (0, 0)),
            pl.BlockSpec((3, 3, 384, 384), lambda n: (0, 0, 0, 0)),
            pl.BlockSpec((1, 384), lambda n: (0, 0)),
            pl.BlockSpec((3, 3, 384, 256), lambda n: (0, 0, 0, 0)),
            pl.BlockSpec((1, 256), lambda n: (0, 0)),
        ],
        out_specs=pl.BlockSpec((1, 6, 8, 256), lambda n: (n, 0, 0, 0)),
        out_shape=jax.ShapeDtypeStruct((batch, 6, 8, 256), BF16),
    )(h, w3s, b3.reshape(1, 384), w4s, b4.reshape(1, 384), w5s,
      b5.reshape(1, 256))

    xt = h[:, :, :6, :].transpose(3, 1, 2, 0).reshape(9216, batch)
    ht = _fc(W6, xt, b6, n_block=512)
    ht = _fc(W7, ht.astype(BF16), b7, n_block=512)
    ht = _fc(W8, ht.astype(BF16), b8, n_block=200)
    return ht.T


# speedup vs baseline: 5.4201x; 1.0632x over previous
import jax
import jax.numpy as jnp
from jax.experimental import pallas as pl

F32 = jnp.float32
BF16 = jnp.bfloat16


def _max3(a, b, c):
    return jnp.maximum(jnp.maximum(a, b), c)


def _pool(v, oh, ow2):
    c = v.shape[2]
    p = v[: 2 * oh].reshape(oh, 2, 2 * ow2, c).max(axis=1)
    q = v[1 : 2 * oh + 1].reshape(oh, 2, 2 * ow2, c)[:, 1]
    hh = jnp.maximum(p, q)
    pv = hh.reshape(oh, ow2, 2, v.shape[2])
    p2 = pv.max(axis=2)
    pp = pv[:, :, 0]
    q2 = jnp.concatenate([pp[:, 1:], pp[:, :1]], axis=1)
    return jnp.maximum(p2, q2)


def _front_body(x_ref, w1_ref, b1_ref, w2_ref, b2_ref, out_ref):
    xf = x_ref[0]
    xcol = jnp.concatenate(
        [xf[dj : dj + 3648, :] for dj in range(3)], axis=1)
    acc = jnp.dot(xcol[0:3520], w1_ref[0], preferred_element_type=F32)
    for di in range(1, 3):
        acc = acc + jnp.dot(xcol[64 * di : 64 * di + 3520], w1_ref[di],
                            preferred_element_type=F32)
    a = acc.reshape(55, 64, 128) + b1_ref[...].reshape(1, 1, 128)
    h1 = _pool(a, 27, 32).astype(BF16)
    col = jax.lax.broadcasted_iota(jnp.int32, (27, 32, 128), 1)
    h1 = jnp.where(col < 27, h1, jnp.zeros((), BF16))
    h1 = h1.reshape(864, 128)
    xext = jnp.concatenate(
        [jnp.zeros((72, 128), BF16), h1, jnp.zeros((72, 128), BF16)], axis=0)
    xcol = jnp.concatenate(
        [xext[6 + dj : 998 + dj, :] for dj in range(5)], axis=1)
    acc = jnp.dot(xcol[0:864], w2_ref[0], preferred_element_type=F32)
    for di in range(1, 5):
        acc = acc + jnp.dot(xcol[32 * di : 32 * di + 864], w2_ref[di],
                            preferred_element_type=F32)
    b = acc.reshape(27, 32, 256) + b2_ref[...].reshape(1, 1, 256)
    out_ref[0] = _pool(b, 13, 16).astype(BF16).reshape(208, 256)


def _conv3x3(v, w_ref, bias, cin):
    r = jax.lax.broadcasted_iota(jnp.int32, (208, cin), 0)
    vm = jnp.where((r % 16) < 13, v, jnp.zeros((), BF16))
    xext = jnp.concatenate(
        [jnp.zeros((24, cin), BF16), vm, jnp.zeros((24, cin), BF16)], axis=0)
    y = bias
    for dj in range(3):
        z = jnp.dot(xext[0:224], w_ref[0, dj], preferred_element_type=F32)
        z = z + jnp.dot(xext[16:240], w_ref[1, dj],
                        preferred_element_type=F32)
        z = z + jnp.dot(xext[32:256], w_ref[2, dj],
                        preferred_element_type=F32)
        y = y + z[7 + dj : 215 + dj]
    return y


def _back_body(x_ref, w3_ref, b3_ref, w4_ref, b4_ref, w5_ref, b5_ref,
               out_ref):
    x = x_ref[0]
    c3 = _conv3x3(x, w3_ref, b3_ref[...].reshape(1, 384), 256)
    c4 = _conv3x3(c3.astype(BF16), w4_ref, b4_ref[...].reshape(1, 384), 384)
    c5 = _conv3x3(c4.astype(BF16), w5_ref, b5_ref[...].reshape(1, 256), 384)
    out_ref[0] = _pool(c5.reshape(13, 16, 256), 6, 8).astype(BF16)


def _fc_body(w_ref, x_ref, b_ref, out_ref):
    out_ref[...] = (
        jnp.dot(w_ref[...].astype(BF16), x_ref[...],
                preferred_element_type=F32)
        + b_ref[...])


def _fc(w, xt, b, n_block):
    n, k = w.shape
    cols = xt.shape[1]
    return pl.pallas_call(
        _fc_body,
        grid=(n // n_block,),
        in_specs=[
            pl.BlockSpec((n_block, k), lambda i: (i, 0)),
            pl.BlockSpec((k, cols), lambda i: (0, 0)),
            pl.BlockSpec((n_block, 1), lambda i: (i, 0)),
        ],
        out_specs=pl.BlockSpec((n_block, cols), lambda i: (i, 0)),
        out_shape=jax.ShapeDtypeStruct((n, cols), F32),
    )(w, xt, b.reshape(n, 1))


@jax.jit
def kernel(x, W1, b1, W2, b2, W3, b3, W4, b4, W5, b5, W6, b6, W7, b7, W8, b8):
    batch = x.shape[0]

    xp = jnp.pad(x.astype(BF16), ((0, 0), (0, 0), (0, 5), (0, 29)))
    xs = xp.reshape(batch, 3, 58, 4, 64, 4).transpose(0, 2, 4, 1, 3, 5)
    xs = xs.reshape(batch, 58 * 64, 48)

    w1p = jnp.pad(W1.astype(BF16), ((0, 32), (0, 0), (0, 1), (0, 1)))
    w1s = w1p.reshape(128, 3, 3, 4, 3, 4).transpose(2, 4, 1, 3, 5, 0)
    w1s = w1s.reshape(3, 3 * 48, 128)
    b1p = jnp.pad(b1, (0, 32)).reshape(1, 128)

    w2s = (jnp.pad(W2.astype(BF16), ((0, 0), (0, 32), (0, 0), (0, 0)))
           .transpose(2, 3, 1, 0).reshape(5, 640, 256))

    def conv_w(w):
        return w.astype(BF16).transpose(2, 3, 1, 0)

    w3s, w4s, w5s = conv_w(W3), conv_w(W4), conv_w(W5)

    h = pl.pallas_call(
        _front_body,
        grid=(batch,),
        in_specs=[
            pl.BlockSpec((1, 58 * 64, 48), lambda n: (n, 0, 0)),
            pl.BlockSpec((3, 3 * 48, 128), lambda n: (0, 0, 0)),
            pl.BlockSpec((1, 128), lambda n: (0, 0)),
            pl.BlockSpec((5, 640, 256), lambda n: (0, 0, 0)),
            pl.BlockSpec((1, 256), lambda n: (0, 0)),
        ],
        out_specs=pl.BlockSpec((1, 208, 256), lambda n: (n, 0, 0)),
        out_shape=jax.ShapeDtypeStruct((batch, 208, 256), BF16),
    )(xs, w1s, b1p, w2s, b2.reshape(1, 256))

    h = pl.pallas_call(
        _back_body,
        grid=(batch,),
        in_specs=[
            pl.BlockSpec((1, 208, 256), lambda n: (n, 0, 0)),
            pl.BlockSpec((3, 3, 256, 384), lambda n: (0, 0, 0, 0)),
            pl.BlockSpec((1, 384), lambda n: (0, 0)),
            pl.BlockSpec((3, 3, 384, 384), lambda n: (0, 0, 0, 0)),
            pl.BlockSpec((1, 384), lambda n: (0, 0)),
            pl.BlockSpec((3, 3, 384, 256), lambda n: (0, 0, 0, 0)),
            pl.BlockSpec((1, 256), lambda n: (0, 0)),
        ],
        out_specs=pl.BlockSpec((1, 6, 8, 256), lambda n: (n, 0, 0, 0)),
        out_shape=jax.ShapeDtypeStruct((batch, 6, 8, 256), BF16),
    )(h, w3s, b3.reshape(1, 384), w4s, b4.reshape(1, 384), w5s,
      b5.reshape(1, 256))

    xt = h[:, :, :6, :].transpose(3, 1, 2, 0).reshape(9216, batch)
    ht = _fc(W6, xt, b6, n_block=512)
    ht = _fc(W7, ht.astype(BF16), b7, n_block=512)
    ht = _fc(W8, ht.astype(BF16), b8, n_block=200)
    return ht.T


# speedup vs baseline: 5.6659x; 1.0453x over previous
import jax
import jax.numpy as jnp
from jax.experimental import pallas as pl

F32 = jnp.float32
BF16 = jnp.bfloat16


def _max3(a, b, c):
    return jnp.maximum(jnp.maximum(a, b), c)


def _pool(v, oh, ow2):
    c = v.shape[2]
    p = v[: 2 * oh].reshape(oh, 2, 2 * ow2, c).max(axis=1)
    q = v[1 : 2 * oh + 1].reshape(oh, 2, 2 * ow2, c)[:, 1]
    hh = jnp.maximum(p, q)
    pv = hh.reshape(oh, ow2, 2, v.shape[2])
    p2 = pv.max(axis=2)
    pp = pv[:, :, 0]
    q2 = jnp.concatenate([pp[:, 1:], pp[:, :1]], axis=1)
    return jnp.maximum(p2, q2)


def _front_body(x_ref, w1_ref, b1_ref, w2_ref, b2_ref, out_ref):
    xf = x_ref[0]
    xcol = jnp.concatenate(
        [xf[dj : dj + 3648, :] for dj in range(3)], axis=1)
    acc = jnp.dot(xcol[0:3520], w1_ref[0], preferred_element_type=F32)
    for di in range(1, 3):
        acc = acc + jnp.dot(xcol[64 * di : 64 * di + 3520], w1_ref[di],
                            preferred_element_type=F32)
    a = acc.reshape(55, 64, 96) + b1_ref[...].reshape(1, 1, 96)
    h1 = _pool(a, 27, 32).astype(BF16)
    col = jax.lax.broadcasted_iota(jnp.int32, (27, 32, 96), 1)
    h1 = jnp.where(col < 27, h1, jnp.zeros((), BF16))
    h1 = h1.reshape(864, 96)
    xext = jnp.concatenate(
        [jnp.zeros((72, 96), BF16), h1, jnp.zeros((72, 96), BF16)], axis=0)
    xcol = jnp.concatenate(
        [xext[6 + dj : 998 + dj, :] for dj in range(5)], axis=1)
    acc = jnp.dot(xcol[0:864], w2_ref[0], preferred_element_type=F32)
    for di in range(1, 5):
        acc = acc + jnp.dot(xcol[32 * di : 32 * di + 864], w2_ref[di],
                            preferred_element_type=F32)
    b = acc.reshape(27, 32, 256) + b2_ref[...].reshape(1, 1, 256)
    out_ref[0] = _pool(b, 13, 16).astype(BF16).reshape(208, 256)


def _conv3x3(v, w_ref, bias, cin):
    r = jax.lax.broadcasted_iota(jnp.int32, (208, cin), 0)
    vm = jnp.where((r % 16) < 13, v, jnp.zeros((), BF16))
    xext = jnp.concatenate(
        [jnp.zeros((24, cin), BF16), vm, jnp.zeros((24, cin), BF16)], axis=0)
    y = bias
    for dj in range(3):
        z = jnp.dot(xext[0:224], w_ref[0, dj], preferred_element_type=F32)
        z = z + jnp.dot(xext[16:240], w_ref[1, dj],
                        preferred_element_type=F32)
        z = z + jnp.dot(xext[32:256], w_ref[2, dj],
                        preferred_element_type=F32)
        y = y + z[7 + dj : 215 + dj]
    return y


def _back_body(x_ref, w3_ref, b3_ref, w4_ref, b4_ref, w5_ref, b5_ref,
               out_ref):
    x = x_ref[0]
    c3 = _conv3x3(x, w3_ref, b3_ref[...].reshape(1, 384), 256)
    c4 = _conv3x3(c3.astype(BF16), w4_ref, b4_ref[...].reshape(1, 384), 384)
    c5 = _conv3x3(c4.astype(BF16), w5_ref, b5_ref[...].reshape(1, 256), 384)
    out_ref[0] = _pool(c5.reshape(13, 16, 256), 6, 8).astype(BF16)


def _fc_body(w_ref, x_ref, b_ref, out_ref):
    out_ref[...] = (
        jnp.dot(w_ref[...].astype(BF16), x_ref[...],
                preferred_element_type=F32)
        + b_ref[...])


def _fc(w, xt, b, n_block):
    n, k = w.shape
    cols = xt.shape[1]
    return pl.pallas_call(
        _fc_body,
        grid=(n // n_block,),
        in_specs=[
            pl.BlockSpec((n_block, k), lambda i: (i, 0)),
            pl.BlockSpec((k, cols), lambda i: (0, 0)),
            pl.BlockSpec((n_block, 1), lambda i: (i, 0)),
        ],
        out_specs=pl.BlockSpec((n_block, cols), lambda i: (i, 0)),
        out_shape=jax.ShapeDtypeStruct((n, cols), F32),
    )(w, xt, b.reshape(n, 1))


@jax.jit
def kernel(x, W1, b1, W2, b2, W3, b3, W4, b4, W5, b5, W6, b6, W7, b7, W8, b8):
    batch = x.shape[0]

    xp = jnp.pad(x.astype(BF16), ((0, 0), (0, 0), (0, 5), (0, 29)))
    xs = xp.reshape(batch, 3, 58, 4, 64, 4).transpose(0, 2, 4, 1, 3, 5)
    xs = xs.reshape(batch, 58 * 64, 48)

    w1p = jnp.pad(W1.astype(BF16), ((0, 0), (0, 0), (0, 1), (0, 1)))
    w1s = w1p.reshape(96, 3, 3, 4, 3, 4).transpose(2, 4, 1, 3, 5, 0)
    w1s = w1s.reshape(3, 3 * 48, 96)
    b1p = b1.reshape(1, 96)

    w2s = W2.astype(BF16).transpose(2, 3, 1, 0).reshape(5, 480, 256)

    def conv_w(w):
        return w.astype(BF16).transpose(2, 3, 1, 0)

    w3s, w4s, w5s = conv_w(W3), conv_w(W4), conv_w(W5)

    h = pl.pallas_call(
        _front_body,
        grid=(batch,),
        in_specs=[
            pl.BlockSpec((1, 58 * 64, 48), lambda n: (n, 0, 0)),
            pl.BlockSpec((3, 3 * 48, 96), lambda n: (0, 0, 0)),
            pl.BlockSpec((1, 96), lambda n: (0, 0)),
            pl.BlockSpec((5, 480, 256), lambda n: (0, 0, 0)),
            pl.BlockSpec((1, 256), lambda n: (0, 0)),
        ],
        out_specs=pl.BlockSpec((1, 208, 256), lambda n: (n, 0, 0)),
        out_shape=jax.ShapeDtypeStruct((batch, 208, 256), BF16),
    )(xs, w1s, b1p, w2s, b2.reshape(1, 256))

    h = pl.pallas_call(
        _back_body,
        grid=(batch,),
        in_specs=[
            pl.BlockSpec((1, 208, 256), lambda n: (n, 0, 0)),
            pl.BlockSpec((3, 3, 256, 384), lambda n: (0, 0, 0, 0)),
            pl.BlockSpec((1, 384), lambda n: (0, 0)),
            pl.BlockSpec((3, 3, 384, 384), lambda n: (0, 0, 0, 0)),
            pl.BlockSpec((1, 384), lambda n: (0, 0)),
            pl.BlockSpec((3, 3, 384, 256), lambda n: (0, 0, 0, 0)),
            pl.BlockSpec((1, 256), lambda n: (0, 0)),
        ],
        out_specs=pl.BlockSpec((1, 6, 8, 256), lambda n: (n, 0, 0, 0)),
        out_shape=jax.ShapeDtypeStruct((batch, 6, 8, 256), BF16),
    )(h, w3s, b3.reshape(1, 384), w4s, b4.reshape(1, 384), w5s,
      b5.reshape(1, 256))

    xt = h[:, :, :6, :].transpose(3, 1, 2, 0).reshape(9216, batch)
    ht = _fc(W6, xt, b6, n_block=512)
    ht = _fc(W7, ht.astype(BF16), b7, n_block=512)
    ht = _fc(W8, ht.astype(BF16), b8, n_block=200)
    return ht.T


# speedup vs baseline: 5.8627x; 1.0347x over previous
import jax
import jax.numpy as jnp
from jax.experimental import pallas as pl
from jax.experimental.pallas import tpu as pltpu

F32 = jnp.float32
BF16 = jnp.bfloat16


def _max3(a, b, c):
    return jnp.maximum(jnp.maximum(a, b), c)


def _pool(v, oh, ow2):
    c = v.shape[2]
    p = v[: 2 * oh].reshape(oh, 2, 2 * ow2, c).max(axis=1)
    q = v[1 : 2 * oh + 1].reshape(oh, 2, 2 * ow2, c)[:, 1]
    hh = jnp.maximum(p, q)
    pv = hh.reshape(oh, ow2, 2, v.shape[2])
    p2 = pv.max(axis=2)
    pp = pv[:, :, 0]
    q2 = jnp.concatenate([pp[:, 1:], pp[:, :1]], axis=1)
    return jnp.maximum(p2, q2)


def _front_body(x_ref, w1_ref, b1_ref, w2_ref, b2_ref, out_ref, s_ref):
    @pl.when(pl.program_id(0) == 0)
    def _():
        ri = jax.lax.broadcasted_iota(jnp.int32, (864, 1728), 0)
        ci = jax.lax.broadcasted_iota(jnp.int32, (864, 1728), 1)
        src = 64 * (ri // 32) + 2 * (ri % 32)
        s_ref[...] = jnp.where(ci == src, 1.0, 0.0).astype(BF16)

    xf = x_ref[0]
    xcol = jnp.concatenate(
        [xf[dj : dj + 3648, :] for dj in range(3)], axis=1)
    acc = jnp.dot(xcol[0:3520], w1_ref[0], preferred_element_type=F32)
    for di in range(1, 3):
        acc = acc + jnp.dot(xcol[64 * di : 64 * di + 3520], w1_ref[di],
                            preferred_element_type=F32)
    p = _max3(acc[0:3392], acc[64:3456], acc[128:3520])
    q = _max3(p[0:3384], p[1:3385], p[2:3386])
    qp = jnp.concatenate([q, jnp.zeros((72, 96), F32)], axis=0)
    qs = qp.reshape(27, 128, 96)[:, 0:64, :].reshape(1728, 96).astype(BF16)
    h1 = (jnp.dot(s_ref[...], qs, preferred_element_type=F32)
          + b1_ref[...])
    rr = jax.lax.broadcasted_iota(jnp.int32, (864, 96), 0)
    h1 = jnp.where((rr % 32) < 27, h1.astype(BF16), jnp.zeros((), BF16))
    xext = jnp.concatenate(
        [jnp.zeros((72, 96), BF16), h1, jnp.zeros((72, 96), BF16)], axis=0)
    xcol = jnp.concatenate(
        [xext[6 + dj : 998 + dj, :] for dj in range(5)], axis=1)
    acc = jnp.dot(xcol[0:864], w2_ref[0], preferred_element_type=F32)
    for di in range(1, 5):
        acc = acc + jnp.dot(xcol[32 * di : 32 * di + 864], w2_ref[di],
                            preferred_element_type=F32)
    b = acc.reshape(27, 32, 256) + b2_ref[...].reshape(1, 1, 256)
    out_ref[0] = _pool(b, 13, 16).astype(BF16).reshape(208, 256)


def _conv3x3(v, w_ref, bias, cin):
    r = jax.lax.broadcasted_iota(jnp.int32, (208, cin), 0)
    vm = jnp.where((r % 16) < 13, v, jnp.zeros((), BF16))
    xext = jnp.concatenate(
        [jnp.zeros((24, cin), BF16), vm, jnp.zeros((24, cin), BF16)], axis=0)
    y = bias
    for dj in range(3):
        z = jnp.dot(xext[0:224], w_ref[0, dj], preferred_element_type=F32)
        z = z + jnp.dot(xext[16:240], w_ref[1, dj],
                        preferred_element_type=F32)
        z = z + jnp.dot(xext[32:256], w_ref[2, dj],
                        preferred_element_type=F32)
        y = y + z[7 + dj : 215 + dj]
    return y


def _back_body(x_ref, w3_ref, b3_ref, w4_ref, b4_ref, w5_ref, b5_ref,
               out_ref):
    x = x_ref[0]
    c3 = _conv3x3(x, w3_ref, b3_ref[...].reshape(1, 384), 256)
    c4 = _conv3x3(c3.astype(BF16), w4_ref, b4_ref[...].reshape(1, 384), 384)
    c5 = _conv3x3(c4.astype(BF16), w5_ref, b5_ref[...].reshape(1, 256), 384)
    out_ref[0] = _pool(c5.reshape(13, 16, 256), 6, 8).astype(BF16)


def _fc_body(w_ref, x_ref, b_ref, out_ref):
    out_ref[...] = (
        jnp.dot(w_ref[...].astype(BF16), x_ref[...],
                preferred_element_type=F32)
        + b_ref[...])


def _fc(w, xt, b, n_block):
    n, k = w.shape
    cols = xt.shape[1]
    return pl.pallas_call(
        _fc_body,
        grid=(n // n_block,),
        in_specs=[
            pl.BlockSpec((n_block, k), lambda i: (i, 0)),
            pl.BlockSpec((k, cols), lambda i: (0, 0)),
            pl.BlockSpec((n_block, 1), lambda i: (i, 0)),
        ],
        out_specs=pl.BlockSpec((n_block, cols), lambda i: (i, 0)),
        out_shape=jax.ShapeDtypeStruct((n, cols), F32),
    )(w, xt, b.reshape(n, 1))


@jax.jit
def kernel(x, W1, b1, W2, b2, W3, b3, W4, b4, W5, b5, W6, b6, W7, b7, W8, b8):
    batch = x.shape[0]

    xp = jnp.pad(x.astype(BF16), ((0, 0), (0, 0), (0, 5), (0, 29)))
    xs = xp.reshape(batch, 3, 58, 4, 64, 4).transpose(0, 2, 4, 1, 3, 5)
    xs = xs.reshape(batch, 58 * 64, 48)

    w1p = jnp.pad(W1.astype(BF16), ((0, 0), (0, 0), (0, 1), (0, 1)))
    w1s = w1p.reshape(96, 3, 3, 4, 3, 4).transpose(2, 4, 1, 3, 5, 0)
    w1s = w1s.reshape(3, 3 * 48, 96)
    b1p = b1.reshape(1, 96)

    w2s = W2.astype(BF16).transpose(2, 3, 1, 0).reshape(5, 480, 256)

    def conv_w(w):
        return w.astype(BF16).transpose(2, 3, 1, 0)

    w3s, w4s, w5s = conv_w(W3), conv_w(W4), conv_w(W5)

    h = pl.pallas_call(
        _front_body,
        grid=(batch,),
        in_specs=[
            pl.BlockSpec((1, 58 * 64, 48), lambda n: (n, 0, 0)),
            pl.BlockSpec((3, 3 * 48, 96), lambda n: (0, 0, 0)),
            pl.BlockSpec((1, 96), lambda n: (0, 0)),
            pl.BlockSpec((5, 480, 256), lambda n: (0, 0, 0)),
            pl.BlockSpec((1, 256), lambda n: (0, 0)),
        ],
        out_specs=pl.BlockSpec((1, 208, 256), lambda n: (n, 0, 0)),
        out_shape=jax.ShapeDtypeStruct((batch, 208, 256), BF16),
        scratch_shapes=[pltpu.VMEM((864, 1728), BF16)],
    )(xs, w1s, b1p, w2s, b2.reshape(1, 256))

    h = pl.pallas_call(
        _back_body,
        grid=(batch,),
        in_specs=[
            pl.BlockSpec((1, 208, 256), lambda n: (n, 0, 0)),
            pl.BlockSpec((3, 3, 256, 384), lambda n: (0, 0, 0, 0)),
            pl.BlockSpec((1, 384), lambda n: (0, 0)),
            pl.BlockSpec((3, 3, 384, 384), lambda n: (0, 0, 0, 0)),
            pl.BlockSpec((1, 384), lambda n: (0, 0)),
            pl.BlockSpec((3, 3, 384, 256), lambda n: (0, 0, 0, 0)),
            pl.BlockSpec((1, 256), lambda n: (0, 0)),
        ],
        out_specs=pl.BlockSpec((1, 6, 8, 256), lambda n: (n, 0, 0, 0)),
        out_shape=jax.ShapeDtypeStruct((batch, 6, 8, 256), BF16),
    )(h, w3s, b3.reshape(1, 384), w4s, b4.reshape(1, 384), w5s,
      b5.reshape(1, 256))

    xt = h[:, :, :6, :].transpose(3, 1, 2, 0).reshape(9216, batch)
    ht = _fc(W6, xt, b6, n_block=512)
    ht = _fc(W7, ht.astype(BF16), b7, n_block=512)
    ht = _fc(W8, ht.astype(BF16), b8, n_block=200)
    return ht.T
